# baseline (speedup 1.0000x reference)
import functools

import jax
import jax.numpy as jnp
from jax import lax
from jax.experimental import pallas as pl
from jax.experimental.pallas import tpu as pltpu
from jax.experimental.pallas import tpu_sc as plsc

B, S, D = 4, 2048, 1024
E = 8
D_FF = 2 * D
N = B * S
BLK = 256
TPAD = N * 2 + E * BLK
NB = TPAD // BLK
RT, LT = 64, 128

NC_SC, NS_SC = 2, 16
NW = NC_SC * NS_SC
TPW = N // NW
CH = 16
NCH = TPW // CH


def _erf(x):
    ax = jnp.abs(x)
    t = 1.0 / (1.0 + 0.3275911 * ax)
    poly = t * (0.254829592 + t * (-0.284496736 + t * (1.421413741
           + t * (-1.453152027 + t * 1.061405429))))
    y = 1.0 - poly * jnp.exp(-ax * ax)
    return jnp.sign(x) * y


def _gelu(x):
    return 0.5 * x * (1.0 + _erf(x * 0.7071067811865476))


def _router_body(x_ref, wr_ref, br_ref, i0_ref, i1_ref, w0_ref, w1_ref):
    logits = jnp.dot(x_ref[...], wr_ref[...],
                     preferred_element_type=jnp.float32) + br_ref[...]
    m = jnp.max(logits, axis=-1, keepdims=True)
    p = jnp.exp(logits - m)
    p = p / jnp.sum(p, axis=-1, keepdims=True)
    lane = lax.broadcasted_iota(jnp.int32, p.shape, 1)
    m1 = jnp.max(p, axis=-1, keepdims=True)
    i1 = jnp.min(jnp.where(p == m1, lane, 127), axis=-1, keepdims=True)
    p2 = jnp.where(lane == i1, -jnp.inf, p)
    m2 = jnp.max(p2, axis=-1, keepdims=True)
    i2 = jnp.min(jnp.where(p2 == m2, lane, 127), axis=-1, keepdims=True)
    denom = m1 + m2 + 1e-8
    i0_ref[...] = i1.astype(jnp.int32)
    i1_ref[...] = i2.astype(jnp.int32)
    w0_ref[...] = jnp.broadcast_to(m1 / denom, w0_ref.shape)
    w1_ref[...] = jnp.broadcast_to(m2 / denom, w1_ref.shape)


def _router(x, Wr, br):
    blk = 1024
    return pl.pallas_call(
        _router_body,
        grid=(N // blk,),
        in_specs=[
            pl.BlockSpec((blk, D), lambda i: (i, 0)),
            pl.BlockSpec((D, E), lambda i: (0, 0)),
            pl.BlockSpec((E,), lambda i: (0,)),
        ],
        out_specs=[
            pl.BlockSpec((blk, 1), lambda i: (i, 0)),
            pl.BlockSpec((blk, 1), lambda i: (i, 0)),
            pl.BlockSpec((blk, 128), lambda i: (i, 0)),
            pl.BlockSpec((blk, 128), lambda i: (i, 0)),
        ],
        out_shape=[
            jax.ShapeDtypeStruct((N, 1), jnp.int32),
            jax.ShapeDtypeStruct((N, 1), jnp.int32),
            jax.ShapeDtypeStruct((N, 128), jnp.float32),
            jax.ShapeDtypeStruct((N, 128), jnp.float32),
        ],
    )(x, Wr, br)


def _shift_lanes(c, sh):
    z = jnp.zeros((RT, sh), jnp.float32)
    return jnp.concatenate([z, c[:, :LT - sh]], axis=1)


def _shift_rows(c, sh):
    z = jnp.zeros((sh, 1), jnp.float32)
    return jnp.concatenate([z, c[:RT - sh]], axis=0)


def _dispatch_body(i0_ref, i1_ref, dst0_ref, dst1_ref, bemeta_ref):
    i0 = i0_ref[...]
    i1 = i1_ref[...]
    excls = []
    counts = []
    for e in range(E):
        oh = ((i0 == e) | (i1 == e)).astype(jnp.float32)
        c = oh
        sh = 1
        while sh < LT:
            c = c + _shift_lanes(c, sh)
            sh *= 2
        row_tot = c[:, LT - 1:LT]
        r = row_tot
        sh = 1
        while sh < RT:
            r = r + _shift_rows(r, sh)
            sh *= 2
        incl = c + (r - row_tot)
        excls.append(incl - oh)
        counts.append(r[RT - 1:RT, 0:1])
    inv_blk = 1.0 / BLK
    acc = jnp.zeros((1, 1), jnp.float32)
    p_rows = []
    pends = []
    for e in range(E):
        nblocks = jnp.floor((counts[e] + (BLK - 1)) * inv_blk)
        p_rows.append(acc * BLK)
        acc = acc + nblocks
        pends.append(acc)
    dst0 = jnp.zeros((RT, LT), jnp.float32)
    dst1 = jnp.zeros((RT, LT), jnp.float32)
    for e in range(E):
        slot = p_rows[e] + excls[e]
        dst0 = dst0 + jnp.where(i0 == e, slot, 0.0)
        dst1 = dst1 + jnp.where(i1 == e, slot, 0.0)
    dst0_ref[...] = dst0.astype(jnp.int32)
    dst1_ref[...] = dst1.astype(jnp.int32)
    iota_l = lax.broadcasted_iota(jnp.int32, (1, LT), 1).astype(jnp.float32)
    be = jnp.zeros((1, LT), jnp.float32)
    for e in range(E):
        be = be + (iota_l >= pends[e]).astype(jnp.float32)
    be = jnp.minimum(be, float(E - 1))
    bemeta = jnp.concatenate(
        [be, jnp.broadcast_to(acc, (1, LT)), jnp.zeros((6, LT), jnp.float32)],
        axis=0)
    bemeta_ref[...] = bemeta.astype(jnp.int32)


def _dispatch(i0r, i1r):
    return pl.pallas_call(
        _dispatch_body,
        grid=(1,),
        in_specs=[
            pl.BlockSpec((RT, LT), lambda i: (0, 0)),
            pl.BlockSpec((RT, LT), lambda i: (0, 0)),
        ],
        out_specs=[
            pl.BlockSpec((RT, LT), lambda i: (0, 0)),
            pl.BlockSpec((RT, LT), lambda i: (0, 0)),
            pl.BlockSpec((8, LT), lambda i: (0, 0)),
        ],
        out_shape=[
            jax.ShapeDtypeStruct((RT, LT), jnp.int32),
            jax.ShapeDtypeStruct((RT, LT), jnp.int32),
            jax.ShapeDtypeStruct((8, LT), jnp.int32),
        ],
    )(i0r, i1r)


def _sc_scatter_fn(x, d0, d1, w0, w1):
    mesh = plsc.VectorSubcoreMesh(core_axis_name="c", subcore_axis_name="s")

    @functools.partial(
        pl.kernel, mesh=mesh,
        out_type=[jax.ShapeDtypeStruct((TPAD, D), jnp.float32),
                  jax.ShapeDtypeStruct((TPAD, 128), jnp.float32)],
        scratch_types=[
            pltpu.VMEM((CH,), jnp.int32),
            pltpu.VMEM((CH,), jnp.int32),
            pltpu.VMEM((CH, D), jnp.float32),
            pltpu.VMEM((CH, 128), jnp.float32),
            pltpu.VMEM((CH, 128), jnp.float32),
            pltpu.SemaphoreType.DMA,
        ],
    )
    def k(x_hbm, d0_hbm, d1_hbm, w0_hbm, w1_hbm, xs_hbm, ws_hbm,
          i0_v, i1_v, rows_v, w0_v, w1_v, sem):
        wid = lax.axis_index("s") * NC_SC + lax.axis_index("c")

        def body(c, carry):
            base = wid * TPW + c * CH
            pltpu.sync_copy(d0_hbm.at[pl.ds(base, CH)], i0_v)
            pltpu.sync_copy(d1_hbm.at[pl.ds(base, CH)], i1_v)
            pltpu.sync_copy(x_hbm.at[pl.ds(base, CH)], rows_v)
            pltpu.sync_copy(w0_hbm.at[pl.ds(base, CH)], w0_v)
            pltpu.sync_copy(w1_hbm.at[pl.ds(base, CH)], w1_v)
            cp0 = pltpu.async_copy(rows_v, xs_hbm.at[i0_v], sem)
            cp1 = pltpu.async_copy(rows_v, xs_hbm.at[i1_v], sem)
            cp2 = pltpu.async_copy(w0_v, ws_hbm.at[i0_v], sem)
            cp3 = pltpu.async_copy(w1_v, ws_hbm.at[i1_v], sem)
            cp0.wait()
            cp1.wait()
            cp2.wait()
            cp3.wait()
            return carry

        lax.fori_loop(0, NCH, body, 0)

    return k(x, d0, d1, w0, w1)


def _gmm_body(be_ref, xs_ref, ws_ref, w1_ref, b1_ref, w2_ref, b2_ref, ys_ref):
    i = pl.program_id(0)

    @pl.when(i < be_ref[LT])
    def _():
        xb = xs_ref[...].astype(jnp.bfloat16)
        h = jnp.dot(xb, w1_ref[0], preferred_element_type=jnp.float32)
        h = _gelu(h + b1_ref[0]).astype(jnp.bfloat16)
        y = jnp.dot(h, w2_ref[0], preferred_element_type=jnp.float32)
        ys_ref[...] = (y + b2_ref[0]) * ws_ref[:, 0:1]


def _gmm(be, xs, ws, W1, b1, W2, b2):
    grid_spec = pltpu.PrefetchScalarGridSpec(
        num_scalar_prefetch=1,
        grid=(NB,),
        in_specs=[
            pl.BlockSpec((BLK, D), lambda i, be: (i, 0)),
            pl.BlockSpec((BLK, 128), lambda i, be: (i, 0)),
            pl.BlockSpec((1, D, D_FF), lambda i, be: (be[i], 0, 0)),
            pl.BlockSpec((1, 1, D_FF), lambda i, be: (be[i], 0, 0)),
            pl.BlockSpec((1, D_FF, D), lambda i, be: (be[i], 0, 0)),
            pl.BlockSpec((1, 1, D), lambda i, be: (be[i], 0, 0)),
        ],
        out_specs=pl.BlockSpec((BLK, D), lambda i, be: (i, 0)),
    )
    return pl.pallas_call(
        _gmm_body,
        grid_spec=grid_spec,
        out_shape=jax.ShapeDtypeStruct((TPAD, D), jnp.float32),
        compiler_params=pltpu.CompilerParams(
            dimension_semantics=("arbitrary",)),
    )(be, xs, ws, W1.astype(jnp.bfloat16), b1.reshape(E, 1, D_FF),
      W2.astype(jnp.bfloat16), b2.reshape(E, 1, D))


def _sc_combine_fn(ys, d0, d1):
    mesh = plsc.VectorSubcoreMesh(core_axis_name="c", subcore_axis_name="s")

    @functools.partial(
        pl.kernel, mesh=mesh,
        out_type=jax.ShapeDtypeStruct((N, D), jnp.float32),
        scratch_types=[
            pltpu.VMEM((CH,), jnp.int32),
            pltpu.VMEM((CH,), jnp.int32),
            pltpu.VMEM((CH, D), jnp.float32),
            pltpu.VMEM((CH, D), jnp.float32),
            pltpu.SemaphoreType.DMA,
        ],
    )
    def k(ys_hbm, d0_hbm, d1_hbm, out_hbm, i0_v, i1_v, r0_v, r1_v, sem):
        wid = lax.axis_index("s") * NC_SC + lax.axis_index("c")

        def body(c, carry):
            base = wid * TPW + c * CH
            pltpu.sync_copy(d0_hbm.at[pl.ds(base, CH)], i0_v)
            pltpu.sync_copy(d1_hbm.at[pl.ds(base, CH)], i1_v)
            g0 = pltpu.async_copy(ys_hbm.at[i0_v], r0_v, sem)
            g1 = pltpu.async_copy(ys_hbm.at[i1_v], r1_v, sem)
            g0.wait()
            g1.wait()

            def brow(rr, cr):
                def bcol(cc, cr2):
                    sl = pl.ds(cc * 16, 16)
                    r0_v[rr, sl] = r0_v[rr, sl] + r1_v[rr, sl]
                    return cr2
                return lax.fori_loop(0, D // 16, bcol, cr, unroll=8)

            lax.fori_loop(0, CH, brow, 0)
            pltpu.sync_copy(r0_v, out_hbm.at[pl.ds(base, CH)])
            return carry

        lax.fori_loop(0, NCH, body, 0)

    return k(ys, d0, d1)


def kernel(hidden_states, Wr, br, W1, b1, W2, b2):
    x = hidden_states.reshape(N, D)
    i0, i1, w0, w1 = _router(x, Wr, br)
    dst0, dst1, bemeta = _dispatch(i0.reshape(RT, LT), i1.reshape(RT, LT))
    be = jnp.concatenate([bemeta[0], bemeta[1, :1]])
    d0 = dst0.reshape(N)
    d1 = dst1.reshape(N)
    xs, ws = _sc_scatter_fn(x, d0, d1, w0, w1)
    ys = _gmm(be, xs, ws, W1, b1, W2, b2)
    out = _sc_combine_fn(ys, d0, d1)
    return out.reshape(B, S, D)


# speedup vs baseline: 1.1834x; 1.1834x over previous
import functools

import jax
import jax.numpy as jnp
from jax import lax
from jax.experimental import pallas as pl
from jax.experimental.pallas import tpu as pltpu
from jax.experimental.pallas import tpu_sc as plsc

B, S, D = 4, 2048, 1024
E = 8
D_FF = 2 * D
N = B * S
BLK = 256
TPAD = N * 2 + E * BLK
NB = TPAD // BLK
RT, LT = 64, 128

NC_SC, NS_SC = 2, 16
NW = NC_SC * NS_SC
TPW = N // NW
CH = 16
NCH = TPW // CH


def _erf(x):
    ax = jnp.abs(x)
    t = 1.0 / (1.0 + 0.3275911 * ax)
    poly = t * (0.254829592 + t * (-0.284496736 + t * (1.421413741
           + t * (-1.453152027 + t * 1.061405429))))
    y = 1.0 - poly * jnp.exp(-ax * ax)
    return jnp.sign(x) * y


def _gelu(x):
    return 0.5 * x * (1.0 + _erf(x * 0.7071067811865476))


def _router_body(x_ref, wr_ref, br_ref, i0_ref, i1_ref, w0_ref, w1_ref):
    logits = jnp.dot(x_ref[...], wr_ref[...],
                     preferred_element_type=jnp.float32) + br_ref[...]
    m = jnp.max(logits, axis=-1, keepdims=True)
    p = jnp.exp(logits - m)
    p = p / jnp.sum(p, axis=-1, keepdims=True)
    lane = lax.broadcasted_iota(jnp.int32, p.shape, 1)
    m1 = jnp.max(p, axis=-1, keepdims=True)
    i1 = jnp.min(jnp.where(p == m1, lane, 127), axis=-1, keepdims=True)
    p2 = jnp.where(lane == i1, -jnp.inf, p)
    m2 = jnp.max(p2, axis=-1, keepdims=True)
    i2 = jnp.min(jnp.where(p2 == m2, lane, 127), axis=-1, keepdims=True)
    denom = m1 + m2 + 1e-8
    i0_ref[...] = i1.astype(jnp.int32)
    i1_ref[...] = i2.astype(jnp.int32)
    w0_ref[...] = jnp.broadcast_to(m1 / denom, w0_ref.shape)
    w1_ref[...] = jnp.broadcast_to(m2 / denom, w1_ref.shape)


def _router(x, Wr, br):
    blk = 1024
    return pl.pallas_call(
        _router_body,
        grid=(N // blk,),
        in_specs=[
            pl.BlockSpec((blk, D), lambda i: (i, 0)),
            pl.BlockSpec((D, E), lambda i: (0, 0)),
            pl.BlockSpec((E,), lambda i: (0,)),
        ],
        out_specs=[
            pl.BlockSpec((blk, 1), lambda i: (i, 0)),
            pl.BlockSpec((blk, 1), lambda i: (i, 0)),
            pl.BlockSpec((blk, 128), lambda i: (i, 0)),
            pl.BlockSpec((blk, 128), lambda i: (i, 0)),
        ],
        out_shape=[
            jax.ShapeDtypeStruct((N, 1), jnp.int32),
            jax.ShapeDtypeStruct((N, 1), jnp.int32),
            jax.ShapeDtypeStruct((N, 128), jnp.float32),
            jax.ShapeDtypeStruct((N, 128), jnp.float32),
        ],
    )(x, Wr, br)


def _shift_lanes(c, sh):
    z = jnp.zeros((RT, sh), jnp.float32)
    return jnp.concatenate([z, c[:, :LT - sh]], axis=1)


def _shift_rows(c, sh):
    z = jnp.zeros((sh, 1), jnp.float32)
    return jnp.concatenate([z, c[:RT - sh]], axis=0)


def _dispatch_body(i0_ref, i1_ref, dst0_ref, dst1_ref, bemeta_ref):
    i0 = i0_ref[...]
    i1 = i1_ref[...]
    tri_l = (lax.broadcasted_iota(jnp.int32, (LT, LT), 0)
             <= lax.broadcasted_iota(jnp.int32, (LT, LT), 1)
             ).astype(jnp.float32)
    tri_r = (lax.broadcasted_iota(jnp.int32, (RT, RT), 1)
             <= lax.broadcasted_iota(jnp.int32, (RT, RT), 0)
             ).astype(jnp.float32)
    excls = []
    counts = []
    for e in range(E):
        oh = ((i0 == e) | (i1 == e)).astype(jnp.float32)
        c = jnp.dot(oh, tri_l, preferred_element_type=jnp.float32)
        row_tot = c[:, LT - 1:LT]
        r = jnp.dot(tri_r, row_tot, preferred_element_type=jnp.float32)
        incl = c + (r - row_tot)
        excls.append(incl - oh)
        counts.append(r[RT - 1:RT, 0:1])
    inv_blk = 1.0 / BLK
    acc = jnp.zeros((1, 1), jnp.float32)
    p_rows = []
    pends = []
    for e in range(E):
        nblocks = jnp.floor((counts[e] + (BLK - 1)) * inv_blk)
        p_rows.append(acc * BLK)
        acc = acc + nblocks
        pends.append(acc)
    dst0 = jnp.zeros((RT, LT), jnp.float32)
    dst1 = jnp.zeros((RT, LT), jnp.float32)
    for e in range(E):
        slot = p_rows[e] + excls[e]
        dst0 = dst0 + jnp.where(i0 == e, slot, 0.0)
        dst1 = dst1 + jnp.where(i1 == e, slot, 0.0)
    dst0_ref[...] = dst0.astype(jnp.int32)
    dst1_ref[...] = dst1.astype(jnp.int32)
    iota_l = lax.broadcasted_iota(jnp.int32, (1, LT), 1).astype(jnp.float32)
    be = jnp.zeros((1, LT), jnp.float32)
    for e in range(E):
        be = be + (iota_l >= pends[e]).astype(jnp.float32)
    be = jnp.minimum(be, float(E - 1))
    bemeta = jnp.concatenate(
        [be, jnp.broadcast_to(acc, (1, LT)), jnp.zeros((6, LT), jnp.float32)],
        axis=0)
    bemeta_ref[...] = bemeta.astype(jnp.int32)


def _dispatch(i0r, i1r):
    return pl.pallas_call(
        _dispatch_body,
        grid=(1,),
        in_specs=[
            pl.BlockSpec((RT, LT), lambda i: (0, 0)),
            pl.BlockSpec((RT, LT), lambda i: (0, 0)),
        ],
        out_specs=[
            pl.BlockSpec((RT, LT), lambda i: (0, 0)),
            pl.BlockSpec((RT, LT), lambda i: (0, 0)),
            pl.BlockSpec((8, LT), lambda i: (0, 0)),
        ],
        out_shape=[
            jax.ShapeDtypeStruct((RT, LT), jnp.int32),
            jax.ShapeDtypeStruct((RT, LT), jnp.int32),
            jax.ShapeDtypeStruct((8, LT), jnp.int32),
        ],
    )(i0r, i1r)


def _sc_scatter_fn(x, d0, d1, w0, w1):
    mesh = plsc.VectorSubcoreMesh(core_axis_name="c", subcore_axis_name="s")

    @functools.partial(
        pl.kernel, mesh=mesh,
        out_type=[jax.ShapeDtypeStruct((TPAD, D), jnp.float32),
                  jax.ShapeDtypeStruct((TPAD, 128), jnp.float32)],
        scratch_types=[
            pltpu.VMEM((TPW,), jnp.int32),
            pltpu.VMEM((TPW,), jnp.int32),
            pltpu.VMEM((2, CH, D), jnp.float32),
            pltpu.VMEM((2, CH, 128), jnp.float32),
            pltpu.VMEM((2, CH, 128), jnp.float32),
            pltpu.SemaphoreType.DMA,
            pltpu.SemaphoreType.DMA,
            pltpu.SemaphoreType.DMA,
            pltpu.SemaphoreType.DMA,
        ],
    )
    def k(x_hbm, d0_hbm, d1_hbm, w0_hbm, w1_hbm, xs_hbm, ws_hbm,
          i0a_v, i1a_v, rows_v, w0_v, w1_v, lsem0, lsem1, ssem0, ssem1):
        wid = lax.axis_index("s") * NC_SC + lax.axis_index("c")
        wbase = wid * TPW
        pltpu.sync_copy(d0_hbm.at[pl.ds(wbase, TPW)], i0a_v)
        pltpu.sync_copy(d1_hbm.at[pl.ds(wbase, TPW)], i1a_v)
        lsems = [lsem0, lsem1]
        ssems = [ssem0, ssem1]

        def issue_loads(c, b):
            base = wbase + c * CH
            return [pltpu.async_copy(x_hbm.at[pl.ds(base, CH)],
                                     rows_v.at[b], lsems[b]),
                    pltpu.async_copy(w0_hbm.at[pl.ds(base, CH)],
                                     w0_v.at[b], lsems[b]),
                    pltpu.async_copy(w1_hbm.at[pl.ds(base, CH)],
                                     w1_v.at[b], lsems[b])]

        load_h = {0: issue_loads(0, 0)}
        scat_h = {}
        for c in range(NCH):
            b = c % 2
            if c + 1 < NCH:
                if c >= 1:
                    for h in scat_h.pop(c - 1):
                        h.wait()
                load_h[c + 1] = issue_loads(c + 1, 1 - b)
            for h in load_h.pop(c):
                h.wait()
            iv0 = i0a_v[pl.ds(c * CH, CH)]
            iv1 = i1a_v[pl.ds(c * CH, CH)]
            scat_h[c] = [
                pltpu.async_copy(rows_v.at[b], xs_hbm.at[iv0], ssems[b]),
                pltpu.async_copy(rows_v.at[b], xs_hbm.at[iv1], ssems[b]),
                pltpu.async_copy(w0_v.at[b], ws_hbm.at[iv0], ssems[b]),
                pltpu.async_copy(w1_v.at[b], ws_hbm.at[iv1], ssems[b]),
            ]
        for c in sorted(scat_h):
            for h in scat_h[c]:
                h.wait()

    return k(x, d0, d1, w0, w1)


def _gmm_body(be_ref, xs_ref, ws_ref, w1_ref, b1_ref, w2_ref, b2_ref, ys_ref):
    i = pl.program_id(0)

    @pl.when(i < be_ref[LT])
    def _():
        h = jnp.dot(xs_ref[...], w1_ref[0], preferred_element_type=jnp.float32)
        h = _gelu(h + b1_ref[0])
        y = jnp.dot(h, w2_ref[0], preferred_element_type=jnp.float32)
        ys_ref[...] = (y + b2_ref[0]) * ws_ref[:, 0:1]


def _gmm(be, xs, ws, W1, b1, W2, b2):
    grid_spec = pltpu.PrefetchScalarGridSpec(
        num_scalar_prefetch=1,
        grid=(NB,),
        in_specs=[
            pl.BlockSpec((BLK, D), lambda i, be: (i, 0)),
            pl.BlockSpec((BLK, 128), lambda i, be: (i, 0)),
            pl.BlockSpec((1, D, D_FF), lambda i, be: (be[i], 0, 0)),
            pl.BlockSpec((1, 1, D_FF), lambda i, be: (be[i], 0, 0)),
            pl.BlockSpec((1, D_FF, D), lambda i, be: (be[i], 0, 0)),
            pl.BlockSpec((1, 1, D), lambda i, be: (be[i], 0, 0)),
        ],
        out_specs=pl.BlockSpec((BLK, D), lambda i, be: (i, 0)),
    )
    return pl.pallas_call(
        _gmm_body,
        grid_spec=grid_spec,
        out_shape=jax.ShapeDtypeStruct((TPAD, D), jnp.float32),
        compiler_params=pltpu.CompilerParams(
            dimension_semantics=("arbitrary",)),
    )(be, xs, ws, W1, b1.reshape(E, 1, D_FF), W2, b2.reshape(E, 1, D))


def _sc_combine_fn(ys, d0, d1):
    mesh = plsc.VectorSubcoreMesh(core_axis_name="c", subcore_axis_name="s")

    @functools.partial(
        pl.kernel, mesh=mesh,
        out_type=jax.ShapeDtypeStruct((N, D), jnp.float32),
        scratch_types=[
            pltpu.VMEM((TPW,), jnp.int32),
            pltpu.VMEM((TPW,), jnp.int32),
            pltpu.VMEM((2, CH, D), jnp.float32),
            pltpu.VMEM((2, CH, D), jnp.float32),
            pltpu.SemaphoreType.DMA,
            pltpu.SemaphoreType.DMA,
            pltpu.SemaphoreType.DMA,
            pltpu.SemaphoreType.DMA,
        ],
    )
    def k(ys_hbm, d0_hbm, d1_hbm, out_hbm, i0a_v, i1a_v, r0_v, r1_v,
          gsem0, gsem1, osem0, osem1):
        wid = lax.axis_index("s") * NC_SC + lax.axis_index("c")
        wbase = wid * TPW
        pltpu.sync_copy(d0_hbm.at[pl.ds(wbase, TPW)], i0a_v)
        pltpu.sync_copy(d1_hbm.at[pl.ds(wbase, TPW)], i1a_v)
        gsems = [gsem0, gsem1]
        osems = [osem0, osem1]

        def issue_gathers(c, b):
            iv0 = i0a_v[pl.ds(c * CH, CH)]
            iv1 = i1a_v[pl.ds(c * CH, CH)]
            return [pltpu.async_copy(ys_hbm.at[iv0], r0_v.at[b], gsems[b]),
                    pltpu.async_copy(ys_hbm.at[iv1], r1_v.at[b], gsems[b])]

        gath_h = {0: issue_gathers(0, 0)}
        out_h = {}
        for c in range(NCH):
            b = c % 2
            if c + 1 < NCH:
                if c >= 1:
                    out_h.pop(c - 1).wait()
                gath_h[c + 1] = issue_gathers(c + 1, 1 - b)
            for h in gath_h.pop(c):
                h.wait()

            def brow(rr, cr, b=b):
                def bcol(cc, cr2):
                    sl = pl.ds(cc * 16, 16)
                    r0_v[b, rr, sl] = r0_v[b, rr, sl] + r1_v[b, rr, sl]
                    return cr2
                return lax.fori_loop(0, D // 16, bcol, cr, unroll=8)

            lax.fori_loop(0, CH, brow, 0)
            out_h[c] = pltpu.async_copy(
                r0_v.at[b], out_hbm.at[pl.ds(wbase + c * CH, CH)], osems[b])
        for c in sorted(out_h):
            out_h[c].wait()

    return k(ys, d0, d1)


def kernel(hidden_states, Wr, br, W1, b1, W2, b2):
    x = hidden_states.reshape(N, D)
    i0, i1, w0, w1 = _router(x, Wr, br)
    dst0, dst1, bemeta = _dispatch(i0.reshape(RT, LT), i1.reshape(RT, LT))
    be = jnp.concatenate([bemeta[0], bemeta[1, :1]])
    d0 = dst0.reshape(N)
    d1 = dst1.reshape(N)
    xs, ws = _sc_scatter_fn(x, d0, d1, w0, w1)
    ys = _gmm(be, xs, ws, W1, b1, W2, b2)
    out = _sc_combine_fn(ys, d0, d1)
    return out.reshape(B, S, D)


# speedup vs baseline: 1.2253x; 1.0354x over previous
import functools

import jax
import jax.numpy as jnp
from jax import lax
from jax.experimental import pallas as pl
from jax.experimental.pallas import tpu as pltpu
from jax.experimental.pallas import tpu_sc as plsc

B, S, D = 4, 2048, 1024
E = 8
D_FF = 2 * D
N = B * S
BLK = 256
TPAD = N * 2 + E * BLK
NB = TPAD // BLK
RT, LT = 64, 128

NC_SC, NS_SC = 2, 16
NW = NC_SC * NS_SC
TPW = N // NW
CH = 16
NCH = TPW // CH


def _erf(x):
    ax = jnp.abs(x)
    t = 1.0 / (1.0 + 0.3275911 * ax)
    poly = t * (0.254829592 + t * (-0.284496736 + t * (1.421413741
           + t * (-1.453152027 + t * 1.061405429))))
    y = 1.0 - poly * jnp.exp(-ax * ax)
    return jnp.sign(x) * y


def _gelu(x):
    return 0.5 * x * (1.0 + _erf(x * 0.7071067811865476))


def _router_body(x_ref, wr_ref, br_ref, i0_ref, i1_ref, w0_ref, w1_ref):
    logits = jnp.dot(x_ref[...], wr_ref[...],
                     preferred_element_type=jnp.float32) + br_ref[...]
    m = jnp.max(logits, axis=-1, keepdims=True)
    p = jnp.exp(logits - m)
    p = p / jnp.sum(p, axis=-1, keepdims=True)
    lane = lax.broadcasted_iota(jnp.int32, p.shape, 1)
    m1 = jnp.max(p, axis=-1, keepdims=True)
    i1 = jnp.min(jnp.where(p == m1, lane, 127), axis=-1, keepdims=True)
    p2 = jnp.where(lane == i1, -jnp.inf, p)
    m2 = jnp.max(p2, axis=-1, keepdims=True)
    i2 = jnp.min(jnp.where(p2 == m2, lane, 127), axis=-1, keepdims=True)
    denom = m1 + m2 + 1e-8
    i0_ref[...] = i1.astype(jnp.int32)
    i1_ref[...] = i2.astype(jnp.int32)
    w0_ref[...] = jnp.broadcast_to(m1 / denom, w0_ref.shape)
    w1_ref[...] = jnp.broadcast_to(m2 / denom, w1_ref.shape)


def _router(x, Wr, br):
    blk = 1024
    return pl.pallas_call(
        _router_body,
        grid=(N // blk,),
        in_specs=[
            pl.BlockSpec((blk, D), lambda i: (i, 0)),
            pl.BlockSpec((D, E), lambda i: (0, 0)),
            pl.BlockSpec((E,), lambda i: (0,)),
        ],
        out_specs=[
            pl.BlockSpec((blk, 1), lambda i: (i, 0)),
            pl.BlockSpec((blk, 1), lambda i: (i, 0)),
            pl.BlockSpec((blk, 128), lambda i: (i, 0)),
            pl.BlockSpec((blk, 128), lambda i: (i, 0)),
        ],
        out_shape=[
            jax.ShapeDtypeStruct((N, 1), jnp.int32),
            jax.ShapeDtypeStruct((N, 1), jnp.int32),
            jax.ShapeDtypeStruct((N, 128), jnp.float32),
            jax.ShapeDtypeStruct((N, 128), jnp.float32),
        ],
    )(x, Wr, br)


def _shift_lanes(c, sh):
    z = jnp.zeros((RT, sh), jnp.float32)
    return jnp.concatenate([z, c[:, :LT - sh]], axis=1)


def _shift_rows(c, sh):
    z = jnp.zeros((sh, 1), jnp.float32)
    return jnp.concatenate([z, c[:RT - sh]], axis=0)


def _dispatch_body(i0_ref, i1_ref, dst0_ref, dst1_ref, bemeta_ref):
    i0 = i0_ref[...]
    i1 = i1_ref[...]
    tri_l = (lax.broadcasted_iota(jnp.int32, (LT, LT), 0)
             <= lax.broadcasted_iota(jnp.int32, (LT, LT), 1)
             ).astype(jnp.float32)
    tri_r = (lax.broadcasted_iota(jnp.int32, (RT, RT), 1)
             <= lax.broadcasted_iota(jnp.int32, (RT, RT), 0)
             ).astype(jnp.float32)
    excls = []
    counts = []
    for e in range(E):
        oh = ((i0 == e) | (i1 == e)).astype(jnp.float32)
        c = jnp.dot(oh, tri_l, preferred_element_type=jnp.float32)
        row_tot = c[:, LT - 1:LT]
        r = jnp.dot(tri_r, row_tot, preferred_element_type=jnp.float32)
        incl = c + (r - row_tot)
        excls.append(incl - oh)
        counts.append(r[RT - 1:RT, 0:1])
    inv_blk = 1.0 / BLK
    acc = jnp.zeros((1, 1), jnp.float32)
    p_rows = []
    pends = []
    for e in range(E):
        nblocks = jnp.floor((counts[e] + (BLK - 1)) * inv_blk)
        p_rows.append(acc * BLK)
        acc = acc + nblocks
        pends.append(acc)
    dst0 = jnp.zeros((RT, LT), jnp.float32)
    dst1 = jnp.zeros((RT, LT), jnp.float32)
    for e in range(E):
        slot = p_rows[e] + excls[e]
        dst0 = dst0 + jnp.where(i0 == e, slot, 0.0)
        dst1 = dst1 + jnp.where(i1 == e, slot, 0.0)
    dst0_ref[...] = dst0.astype(jnp.int32)
    dst1_ref[...] = dst1.astype(jnp.int32)
    iota_l = lax.broadcasted_iota(jnp.int32, (1, LT), 1).astype(jnp.float32)
    be = jnp.zeros((1, LT), jnp.float32)
    for e in range(E):
        be = be + (iota_l >= pends[e]).astype(jnp.float32)
    be = jnp.minimum(be, float(E - 1))
    bemeta = jnp.concatenate(
        [be, jnp.broadcast_to(acc, (1, LT)), jnp.zeros((6, LT), jnp.float32)],
        axis=0)
    bemeta_ref[...] = bemeta.astype(jnp.int32)


def _dispatch(i0r, i1r):
    return pl.pallas_call(
        _dispatch_body,
        grid=(1,),
        in_specs=[
            pl.BlockSpec((RT, LT), lambda i: (0, 0)),
            pl.BlockSpec((RT, LT), lambda i: (0, 0)),
        ],
        out_specs=[
            pl.BlockSpec((RT, LT), lambda i: (0, 0)),
            pl.BlockSpec((RT, LT), lambda i: (0, 0)),
            pl.BlockSpec((8, LT), lambda i: (0, 0)),
        ],
        out_shape=[
            jax.ShapeDtypeStruct((RT, LT), jnp.int32),
            jax.ShapeDtypeStruct((RT, LT), jnp.int32),
            jax.ShapeDtypeStruct((8, LT), jnp.int32),
        ],
    )(i0r, i1r)


def _sc_scatter_fn(x, d0, d1, w0, w1):
    mesh = plsc.VectorSubcoreMesh(core_axis_name="c", subcore_axis_name="s")

    @functools.partial(
        pl.kernel, mesh=mesh,
        out_type=[jax.ShapeDtypeStruct((TPAD, D), jnp.float32),
                  jax.ShapeDtypeStruct((TPAD, 128), jnp.float32)],
        scratch_types=[
            pltpu.VMEM((TPW,), jnp.int32),
            pltpu.VMEM((TPW,), jnp.int32),
            pltpu.VMEM((2, CH, D), jnp.float32),
            pltpu.VMEM((2, CH, 128), jnp.float32),
            pltpu.VMEM((2, CH, 128), jnp.float32),
            pltpu.SemaphoreType.DMA,
            pltpu.SemaphoreType.DMA,
            pltpu.SemaphoreType.DMA,
            pltpu.SemaphoreType.DMA,
        ],
    )
    def k(x_hbm, d0_hbm, d1_hbm, w0_hbm, w1_hbm, xs_hbm, ws_hbm,
          i0a_v, i1a_v, rows_v, w0_v, w1_v, lsem0, lsem1, ssem0, ssem1):
        wid = lax.axis_index("s") * NC_SC + lax.axis_index("c")
        wbase = wid * TPW
        pltpu.sync_copy(d0_hbm.at[pl.ds(wbase, TPW)], i0a_v)
        pltpu.sync_copy(d1_hbm.at[pl.ds(wbase, TPW)], i1a_v)
        lsems = [lsem0, lsem1]
        ssems = [ssem0, ssem1]

        def issue_loads(c, b):
            base = wbase + c * CH
            return [pltpu.async_copy(x_hbm.at[pl.ds(base, CH)],
                                     rows_v.at[b], lsems[b]),
                    pltpu.async_copy(w0_hbm.at[pl.ds(base, CH)],
                                     w0_v.at[b], lsems[b]),
                    pltpu.async_copy(w1_hbm.at[pl.ds(base, CH)],
                                     w1_v.at[b], lsems[b])]

        load_h = {0: issue_loads(0, 0)}
        scat_h = {}
        for c in range(NCH):
            b = c % 2
            if c + 1 < NCH:
                if c >= 1:
                    for h in scat_h.pop(c - 1):
                        h.wait()
                load_h[c + 1] = issue_loads(c + 1, 1 - b)
            for h in load_h.pop(c):
                h.wait()
            iv0 = i0a_v[pl.ds(c * CH, CH)]
            iv1 = i1a_v[pl.ds(c * CH, CH)]
            scat_h[c] = [
                pltpu.async_copy(rows_v.at[b], xs_hbm.at[iv0], ssems[b]),
                pltpu.async_copy(rows_v.at[b], xs_hbm.at[iv1], ssems[b]),
                pltpu.async_copy(w0_v.at[b], ws_hbm.at[iv0], ssems[b]),
                pltpu.async_copy(w1_v.at[b], ws_hbm.at[iv1], ssems[b]),
            ]
        for c in sorted(scat_h):
            for h in scat_h[c]:
                h.wait()

    return k(x, d0, d1, w0, w1)


def _gmm_body(be_ref, xs_ref, ws_ref, w1_ref, b1_ref, w2_ref, b2_ref, ys_ref):
    i = pl.program_id(0)

    @pl.when(i < be_ref[LT])
    def _():
        h = jnp.dot(xs_ref[...], w1_ref[0], preferred_element_type=jnp.float32)
        h = _gelu(h + b1_ref[0])
        y = jnp.dot(h, w2_ref[0], preferred_element_type=jnp.float32)
        ys_ref[...] = (y + b2_ref[0]) * ws_ref[:, 0:1]


def _gmm(be, xs, ws, W1, b1, W2, b2):
    grid_spec = pltpu.PrefetchScalarGridSpec(
        num_scalar_prefetch=1,
        grid=(NB,),
        in_specs=[
            pl.BlockSpec((BLK, D), lambda i, be: (i, 0)),
            pl.BlockSpec((BLK, 128), lambda i, be: (i, 0)),
            pl.BlockSpec((1, D, D_FF), lambda i, be: (be[i], 0, 0)),
            pl.BlockSpec((1, 1, D_FF), lambda i, be: (be[i], 0, 0)),
            pl.BlockSpec((1, D_FF, D), lambda i, be: (be[i], 0, 0)),
            pl.BlockSpec((1, 1, D), lambda i, be: (be[i], 0, 0)),
        ],
        out_specs=pl.BlockSpec((BLK, D), lambda i, be: (i, 0)),
    )
    return pl.pallas_call(
        _gmm_body,
        grid_spec=grid_spec,
        out_shape=jax.ShapeDtypeStruct((TPAD, D), jnp.float32),
        compiler_params=pltpu.CompilerParams(
            dimension_semantics=("arbitrary",)),
    )(be, xs, ws, W1, b1.reshape(E, 1, D_FF), W2, b2.reshape(E, 1, D))


def _sc_combine_fn(ys, d0, d1):
    mesh = plsc.VectorSubcoreMesh(core_axis_name="c", subcore_axis_name="s")

    @functools.partial(
        pl.kernel, mesh=mesh,
        out_type=jax.ShapeDtypeStruct((N, D), jnp.float32),
        scratch_types=[
            pltpu.VMEM((TPW,), jnp.int32),
            pltpu.VMEM((TPW,), jnp.int32),
            pltpu.VMEM((2, CH, D), jnp.float32),
            pltpu.VMEM((2, CH, D), jnp.float32),
            pltpu.SemaphoreType.DMA,
            pltpu.SemaphoreType.DMA,
            pltpu.SemaphoreType.DMA,
            pltpu.SemaphoreType.DMA,
        ],
    )
    def k(ys_hbm, d0_hbm, d1_hbm, out_hbm, i0a_v, i1a_v, r0_v, r1_v,
          gsem0, gsem1, osem0, osem1):
        wid = lax.axis_index("s") * NC_SC + lax.axis_index("c")
        wbase = wid * TPW
        pltpu.sync_copy(d0_hbm.at[pl.ds(wbase, TPW)], i0a_v)
        pltpu.sync_copy(d1_hbm.at[pl.ds(wbase, TPW)], i1a_v)
        gsems = [gsem0, gsem1]
        osems = [osem0, osem1]

        def issue_gathers(c, b):
            iv0 = i0a_v[pl.ds(c * CH, CH)]
            iv1 = i1a_v[pl.ds(c * CH, CH)]
            return [pltpu.async_copy(ys_hbm.at[iv0], r0_v.at[b], gsems[b]),
                    pltpu.async_copy(ys_hbm.at[iv1], r1_v.at[b], gsems[b])]

        gath_h = {0: issue_gathers(0, 0)}
        out_h = {}
        for c in range(NCH):
            b = c % 2
            if c + 1 < NCH:
                if c >= 1:
                    out_h.pop(c - 1).wait()
                gath_h[c + 1] = issue_gathers(c + 1, 1 - b)
            for h in gath_h.pop(c):
                h.wait()

            def brow(rr, cr, b=b):
                def bcol(cc, cr2):
                    sl = pl.ds(cc * 16, 16)
                    plsc.addupdate(r0_v.at[b, rr, sl], r1_v[b, rr, sl])
                    return cr2
                return lax.fori_loop(0, D // 16, bcol, cr, unroll=8)

            lax.fori_loop(0, CH, brow, 0)
            out_h[c] = pltpu.async_copy(
                r0_v.at[b], out_hbm.at[pl.ds(wbase + c * CH, CH)], osems[b])
        for c in sorted(out_h):
            out_h[c].wait()

    return k(ys, d0, d1)


def kernel(hidden_states, Wr, br, W1, b1, W2, b2):
    x = hidden_states.reshape(N, D)
    i0, i1, w0, w1 = _router(x, Wr, br)
    dst0, dst1, bemeta = _dispatch(i0.reshape(RT, LT), i1.reshape(RT, LT))
    be = jnp.concatenate([bemeta[0], bemeta[1, :1]])
    d0 = dst0.reshape(N)
    d1 = dst1.reshape(N)
    xs, ws = _sc_scatter_fn(x, d0, d1, w0, w1)
    ys = _gmm(be, xs, ws, W1, b1, W2, b2)
    out = _sc_combine_fn(ys, d0, d1)
    return out.reshape(B, S, D)


# speedup vs baseline: 1.3760x; 1.1231x over previous
import functools

import jax
import jax.numpy as jnp
from jax import lax
from jax.experimental import pallas as pl
from jax.experimental.pallas import tpu as pltpu
from jax.experimental.pallas import tpu_sc as plsc

B, S, D = 4, 2048, 1024
E = 8
D_FF = 2 * D
N = B * S
BLK = 512
TPAD = N * 2 + E * BLK
NB = TPAD // BLK
RT, LT = 64, 128

NC_SC, NS_SC = 2, 16
NW = NC_SC * NS_SC
TPW = N // NW
CH = 16
NCH = TPW // CH


def _erf(x):
    ax = jnp.abs(x)
    t = 1.0 / (1.0 + 0.3275911 * ax)
    poly = t * (0.254829592 + t * (-0.284496736 + t * (1.421413741
           + t * (-1.453152027 + t * 1.061405429))))
    y = 1.0 - poly * jnp.exp(-ax * ax)
    return jnp.sign(x) * y


def _gelu(x):
    return 0.5 * x * (1.0 + _erf(x * 0.7071067811865476))


def _router_body(x_ref, wr_ref, br_ref, i0_ref, i1_ref, w0_ref, w1_ref):
    logits = jnp.dot(x_ref[...], wr_ref[...],
                     preferred_element_type=jnp.float32) + br_ref[...]
    m = jnp.max(logits, axis=-1, keepdims=True)
    p = jnp.exp(logits - m)
    p = p / jnp.sum(p, axis=-1, keepdims=True)
    lane = lax.broadcasted_iota(jnp.int32, p.shape, 1)
    m1 = jnp.max(p, axis=-1, keepdims=True)
    i1 = jnp.min(jnp.where(p == m1, lane, 127), axis=-1, keepdims=True)
    p2 = jnp.where(lane == i1, -jnp.inf, p)
    m2 = jnp.max(p2, axis=-1, keepdims=True)
    i2 = jnp.min(jnp.where(p2 == m2, lane, 127), axis=-1, keepdims=True)
    denom = m1 + m2 + 1e-8
    i0_ref[...] = i1.astype(jnp.int32)
    i1_ref[...] = i2.astype(jnp.int32)
    w0_ref[...] = jnp.broadcast_to(m1 / denom, w0_ref.shape)
    w1_ref[...] = jnp.broadcast_to(m2 / denom, w1_ref.shape)


def _router(x, Wr, br):
    blk = 1024
    return pl.pallas_call(
        _router_body,
        grid=(N // blk,),
        in_specs=[
            pl.BlockSpec((blk, D), lambda i: (i, 0)),
            pl.BlockSpec((D, E), lambda i: (0, 0)),
            pl.BlockSpec((E,), lambda i: (0,)),
        ],
        out_specs=[
            pl.BlockSpec((blk, 1), lambda i: (i, 0)),
            pl.BlockSpec((blk, 1), lambda i: (i, 0)),
            pl.BlockSpec((blk, 128), lambda i: (i, 0)),
            pl.BlockSpec((blk, 128), lambda i: (i, 0)),
        ],
        out_shape=[
            jax.ShapeDtypeStruct((N, 1), jnp.int32),
            jax.ShapeDtypeStruct((N, 1), jnp.int32),
            jax.ShapeDtypeStruct((N, 128), jnp.float32),
            jax.ShapeDtypeStruct((N, 128), jnp.float32),
        ],
    )(x, Wr, br)


def _shift_lanes(c, sh):
    z = jnp.zeros((RT, sh), jnp.float32)
    return jnp.concatenate([z, c[:, :LT - sh]], axis=1)


def _shift_rows(c, sh):
    z = jnp.zeros((sh, 1), jnp.float32)
    return jnp.concatenate([z, c[:RT - sh]], axis=0)


def _dispatch_body(i0_ref, i1_ref, dst0_ref, dst1_ref, bemeta_ref):
    i0 = i0_ref[...]
    i1 = i1_ref[...]
    tri_l = (lax.broadcasted_iota(jnp.int32, (LT, LT), 0)
             <= lax.broadcasted_iota(jnp.int32, (LT, LT), 1)
             ).astype(jnp.float32)
    tri_r = (lax.broadcasted_iota(jnp.int32, (RT, RT), 1)
             <= lax.broadcasted_iota(jnp.int32, (RT, RT), 0)
             ).astype(jnp.float32)
    excls = []
    counts = []
    for e in range(E):
        oh = ((i0 == e) | (i1 == e)).astype(jnp.float32)
        c = jnp.dot(oh, tri_l, preferred_element_type=jnp.float32)
        row_tot = c[:, LT - 1:LT]
        r = jnp.dot(tri_r, row_tot, preferred_element_type=jnp.float32)
        incl = c + (r - row_tot)
        excls.append(incl - oh)
        counts.append(r[RT - 1:RT, 0:1])
    inv_blk = 1.0 / BLK
    acc = jnp.zeros((1, 1), jnp.float32)
    p_rows = []
    pends = []
    for e in range(E):
        nblocks = jnp.floor((counts[e] + (BLK - 1)) * inv_blk)
        p_rows.append(acc * BLK)
        acc = acc + nblocks
        pends.append(acc)
    dst0 = jnp.zeros((RT, LT), jnp.float32)
    dst1 = jnp.zeros((RT, LT), jnp.float32)
    for e in range(E):
        slot = p_rows[e] + excls[e]
        dst0 = dst0 + jnp.where(i0 == e, slot, 0.0)
        dst1 = dst1 + jnp.where(i1 == e, slot, 0.0)
    dst0_ref[...] = dst0.astype(jnp.int32)
    dst1_ref[...] = dst1.astype(jnp.int32)
    iota_l = lax.broadcasted_iota(jnp.int32, (1, LT), 1).astype(jnp.float32)
    be = jnp.zeros((1, LT), jnp.float32)
    for e in range(E):
        be = be + (iota_l >= pends[e]).astype(jnp.float32)
    be = jnp.minimum(be, float(E - 1))
    bemeta = jnp.concatenate(
        [be, jnp.broadcast_to(acc, (1, LT)), jnp.zeros((6, LT), jnp.float32)],
        axis=0)
    bemeta_ref[...] = bemeta.astype(jnp.int32)


def _dispatch(i0r, i1r):
    return pl.pallas_call(
        _dispatch_body,
        grid=(1,),
        in_specs=[
            pl.BlockSpec((RT, LT), lambda i: (0, 0)),
            pl.BlockSpec((RT, LT), lambda i: (0, 0)),
        ],
        out_specs=[
            pl.BlockSpec((RT, LT), lambda i: (0, 0)),
            pl.BlockSpec((RT, LT), lambda i: (0, 0)),
            pl.BlockSpec((8, LT), lambda i: (0, 0)),
        ],
        out_shape=[
            jax.ShapeDtypeStruct((RT, LT), jnp.int32),
            jax.ShapeDtypeStruct((RT, LT), jnp.int32),
            jax.ShapeDtypeStruct((8, LT), jnp.int32),
        ],
    )(i0r, i1r)


def _sc_scatter_fn(x, d0, d1, w0, w1):
    mesh = plsc.VectorSubcoreMesh(core_axis_name="c", subcore_axis_name="s")

    @functools.partial(
        pl.kernel, mesh=mesh,
        out_type=[jax.ShapeDtypeStruct((TPAD, D), jnp.float32),
                  jax.ShapeDtypeStruct((TPAD, 128), jnp.float32)],
        scratch_types=[
            pltpu.VMEM((TPW,), jnp.int32),
            pltpu.VMEM((TPW,), jnp.int32),
            pltpu.VMEM((2, CH, D), jnp.float32),
            pltpu.VMEM((2, CH, 128), jnp.float32),
            pltpu.VMEM((2, CH, 128), jnp.float32),
            pltpu.SemaphoreType.DMA,
            pltpu.SemaphoreType.DMA,
            pltpu.SemaphoreType.DMA,
            pltpu.SemaphoreType.DMA,
        ],
    )
    def k(x_hbm, d0_hbm, d1_hbm, w0_hbm, w1_hbm, xs_hbm, ws_hbm,
          i0a_v, i1a_v, rows_v, w0_v, w1_v, lsem0, lsem1, ssem0, ssem1):
        wid = lax.axis_index("s") * NC_SC + lax.axis_index("c")
        wbase = wid * TPW
        pltpu.sync_copy(d0_hbm.at[pl.ds(wbase, TPW)], i0a_v)
        pltpu.sync_copy(d1_hbm.at[pl.ds(wbase, TPW)], i1a_v)
        lsems = [lsem0, lsem1]
        ssems = [ssem0, ssem1]

        def issue_loads(c, b):
            base = wbase + c * CH
            return [pltpu.async_copy(x_hbm.at[pl.ds(base, CH)],
                                     rows_v.at[b], lsems[b]),
                    pltpu.async_copy(w0_hbm.at[pl.ds(base, CH)],
                                     w0_v.at[b], lsems[b]),
                    pltpu.async_copy(w1_hbm.at[pl.ds(base, CH)],
                                     w1_v.at[b], lsems[b])]

        load_h = {0: issue_loads(0, 0)}
        scat_h = {}
        for c in range(NCH):
            b = c % 2
            if c + 1 < NCH:
                if c >= 1:
                    for h in scat_h.pop(c - 1):
                        h.wait()
                load_h[c + 1] = issue_loads(c + 1, 1 - b)
            for h in load_h.pop(c):
                h.wait()
            iv0 = i0a_v[pl.ds(c * CH, CH)]
            iv1 = i1a_v[pl.ds(c * CH, CH)]
            scat_h[c] = [
                pltpu.async_copy(rows_v.at[b], xs_hbm.at[iv0], ssems[b]),
                pltpu.async_copy(rows_v.at[b], xs_hbm.at[iv1], ssems[b]),
                pltpu.async_copy(w0_v.at[b], ws_hbm.at[iv0], ssems[b]),
                pltpu.async_copy(w1_v.at[b], ws_hbm.at[iv1], ssems[b]),
            ]
        for c in sorted(scat_h):
            for h in scat_h[c]:
                h.wait()

    return k(x, d0, d1, w0, w1)


def _gmm_body(be_ref, xs_ref, ws_ref, w1_ref, b1_ref, w2_ref, b2_ref, ys_ref):
    i = pl.program_id(0)

    @pl.when(i < be_ref[LT])
    def _():
        h = jnp.dot(xs_ref[...], w1_ref[0], preferred_element_type=jnp.float32)
        h = _gelu(h + b1_ref[0])
        y = jnp.dot(h, w2_ref[0], preferred_element_type=jnp.float32)
        ys_ref[...] = (y + b2_ref[0]) * ws_ref[:, 0:1]


def _gmm(be, xs, ws, W1, b1, W2, b2):
    grid_spec = pltpu.PrefetchScalarGridSpec(
        num_scalar_prefetch=1,
        grid=(NB,),
        in_specs=[
            pl.BlockSpec((BLK, D), lambda i, be: (i, 0)),
            pl.BlockSpec((BLK, 128), lambda i, be: (i, 0)),
            pl.BlockSpec((1, D, D_FF), lambda i, be: (be[i], 0, 0)),
            pl.BlockSpec((1, 1, D_FF), lambda i, be: (be[i], 0, 0)),
            pl.BlockSpec((1, D_FF, D), lambda i, be: (be[i], 0, 0)),
            pl.BlockSpec((1, 1, D), lambda i, be: (be[i], 0, 0)),
        ],
        out_specs=pl.BlockSpec((BLK, D), lambda i, be: (i, 0)),
    )
    return pl.pallas_call(
        _gmm_body,
        grid_spec=grid_spec,
        out_shape=jax.ShapeDtypeStruct((TPAD, D), jnp.float32),
        compiler_params=pltpu.CompilerParams(
            dimension_semantics=("arbitrary",)),
    )(be, xs, ws, W1, b1.reshape(E, 1, D_FF), W2, b2.reshape(E, 1, D))


def _sc_combine_fn(ys, d0, d1):
    mesh = plsc.VectorSubcoreMesh(core_axis_name="c", subcore_axis_name="s")

    @functools.partial(
        pl.kernel, mesh=mesh,
        out_type=jax.ShapeDtypeStruct((N, D), jnp.float32),
        scratch_types=[
            pltpu.VMEM((TPW,), jnp.int32),
            pltpu.VMEM((TPW,), jnp.int32),
            pltpu.VMEM((2, CH, D), jnp.float32),
            pltpu.VMEM((2, CH, D), jnp.float32),
            pltpu.SemaphoreType.DMA,
            pltpu.SemaphoreType.DMA,
            pltpu.SemaphoreType.DMA,
            pltpu.SemaphoreType.DMA,
        ],
    )
    def k(ys_hbm, d0_hbm, d1_hbm, out_hbm, i0a_v, i1a_v, r0_v, r1_v,
          gsem0, gsem1, osem0, osem1):
        wid = lax.axis_index("s") * NC_SC + lax.axis_index("c")
        wbase = wid * TPW
        pltpu.sync_copy(d0_hbm.at[pl.ds(wbase, TPW)], i0a_v)
        pltpu.sync_copy(d1_hbm.at[pl.ds(wbase, TPW)], i1a_v)
        gsems = [gsem0, gsem1]
        osems = [osem0, osem1]

        def issue_gathers(c, b):
            iv0 = i0a_v[pl.ds(c * CH, CH)]
            iv1 = i1a_v[pl.ds(c * CH, CH)]
            return [pltpu.async_copy(ys_hbm.at[iv0], r0_v.at[b], gsems[b]),
                    pltpu.async_copy(ys_hbm.at[iv1], r1_v.at[b], gsems[b])]

        gath_h = {0: issue_gathers(0, 0)}
        out_h = {}
        for c in range(NCH):
            b = c % 2
            if c + 1 < NCH:
                if c >= 1:
                    out_h.pop(c - 1).wait()
                gath_h[c + 1] = issue_gathers(c + 1, 1 - b)
            for h in gath_h.pop(c):
                h.wait()

            def brow(rr, cr, b=b):
                def bcol(cc, cr2):
                    sl = pl.ds(cc * 16, 16)
                    plsc.addupdate(r0_v.at[b, rr, sl], r1_v[b, rr, sl])
                    return cr2
                return lax.fori_loop(0, D // 16, bcol, cr, unroll=8)

            lax.fori_loop(0, CH, brow, 0)
            out_h[c] = pltpu.async_copy(
                r0_v.at[b], out_hbm.at[pl.ds(wbase + c * CH, CH)], osems[b])
        for c in sorted(out_h):
            out_h[c].wait()

    return k(ys, d0, d1)


def kernel(hidden_states, Wr, br, W1, b1, W2, b2):
    x = hidden_states.reshape(N, D)
    i0, i1, w0, w1 = _router(x, Wr, br)
    dst0, dst1, bemeta = _dispatch(i0.reshape(RT, LT), i1.reshape(RT, LT))
    be = jnp.concatenate([bemeta[0], bemeta[1, :1]])
    d0 = dst0.reshape(N)
    d1 = dst1.reshape(N)
    xs, ws = _sc_scatter_fn(x, d0, d1, w0, w1)
    ys = _gmm(be, xs, ws, W1, b1, W2, b2)
    out = _sc_combine_fn(ys, d0, d1)
    return out.reshape(B, S, D)
